# baseline (speedup 1.0000x reference)
import jax
import jax.numpy as jnp
from jax import lax
from jax.experimental import pallas as pl
from jax.experimental.pallas import tpu as pltpu
from jax.experimental.pallas import tpu_sc as plsc

_N = 10000
_E = 320000
_H = 128
_KAPPA = 0.99
_ITERS = 30

_CHUNK = 128
_NCH = 80
_EPT = _NCH * _CHUNK
_EPAD = _EPT * 32
_RPT = _N // 16

_NP = 10240
_CPT = _NP // 16
_REPT = _E // 16

_f32 = jnp.float32


_NSC = 4
_SCCH = _NCH // _NSC
_SCE = _SCCH * _CHUNK


def _sc_spmm_body(y_hbm, src_hbm, dst_hbm, val_hbm, out_hbm,
                  acc, ga, gb, se0, se1, de0, de1, ve0, ve1,
                  esem0, esem1, gsa, gsb):
    c = lax.axis_index("c")
    s = lax.axis_index("s")
    wid = c * 16 + s
    zv = jnp.zeros((16,), _f32)

    def zrow(r, _):
        for f in range(8):
            ga[r, 16 * f:16 * (f + 1)] = zv
        return 0
    lax.fori_loop(0, _CHUNK, zrow, 0)
    row0 = s * 632
    nch8 = jnp.where(s < 15, 79, 65)

    def zset(j, _):
        pltpu.sync_copy(ga.at[pl.ds(0, 8)], acc.at[pl.ds(row0 + 8 * j, 8)])
        return 0
    lax.fori_loop(0, nch8, zset, 0)
    plsc.subcore_barrier()

    def weight(gbuf, veb, ch):
        def wgrp(g, _):
            val16 = veb[pl.ds(_CHUNK * ch + 16 * g, 16)]
            for lane in range(16):
                e = 16 * g + lane
                v = val16[lane]
                for f in range(8):
                    gbuf[e, 16 * f:16 * (f + 1)] = (
                        gbuf[e, 16 * f:16 * (f + 1)] * v)
            return 0
        lax.fori_loop(0, _CHUNK // 16, wgrp, 0)

    ebufs = ((se0, de0, ve0, esem0), (se1, de1, ve1, esem1))

    def e_issue(sci, sb):
        seb, deb, veb, esem = ebufs[sb]
        pltpu.async_copy(src_hbm.at[wid, sci], seb, esem)
        pltpu.async_copy(dst_hbm.at[wid, sci], deb, esem)
        pltpu.async_copy(val_hbm.at[wid, sci], veb, esem)

    def e_wait(sci, sb):
        seb, deb, veb, esem = ebufs[sb]
        pltpu.make_async_copy(src_hbm.at[wid, sci], seb, esem).wait()
        pltpu.make_async_copy(dst_hbm.at[wid, sci], deb, esem).wait()
        pltpu.make_async_copy(val_hbm.at[wid, sci], veb, esem).wait()

    e_issue(0, 0)
    e_issue(1, 1)

    def outer(i, _):
        for sb in range(2):
            sci = 2 * i + sb
            seb, deb, veb, esem = ebufs[sb]
            e_wait(sci, sb)
            pltpu.async_copy(y_hbm.at[seb.at[pl.ds(0, _CHUNK)]], ga, gsa)

            def pair(p, _):
                k0 = 2 * p
                k1 = k0 + 1
                pltpu.async_copy(
                    y_hbm.at[seb.at[pl.ds(_CHUNK * k1, _CHUNK)]], gb, gsb)
                pltpu.make_async_copy(
                    y_hbm.at[seb.at[pl.ds(_CHUNK * k0, _CHUNK)]],
                    ga, gsa).wait()
                weight(ga, veb, k0)
                pltpu.sync_copy(ga, acc.at[deb.at[k0]], add=True)

                @pl.when(p < _SCCH // 2 - 1)
                def _():
                    pltpu.async_copy(
                        y_hbm.at[seb.at[pl.ds(_CHUNK * (k0 + 2), _CHUNK)]],
                        ga, gsa)
                pltpu.make_async_copy(
                    y_hbm.at[seb.at[pl.ds(_CHUNK * k1, _CHUNK)]],
                    gb, gsb).wait()
                weight(gb, veb, k1)
                pltpu.sync_copy(gb, acc.at[deb.at[k1]], add=True)
                return 0
            lax.fori_loop(0, _SCCH // 2, pair, 0)

            @pl.when(i < _NSC // 2 - 1)
            def _():
                e_issue(sci + 2, sb)
        return 0
    lax.fori_loop(0, _NSC // 2, outer, 0)
    plsc.subcore_barrier()

    def wout(j, _):
        o = row0 + 8 * j
        pltpu.sync_copy(acc.at[pl.ds(o, 8)], out_hbm.at[c, pl.ds(o, 8)])
        return 0
    lax.fori_loop(0, nch8, wout, 0)


def _sc_spmm(y, srcp, dstp, valp):
    mesh = plsc.VectorSubcoreMesh(core_axis_name="c", subcore_axis_name="s")
    fn = pl.kernel(
        _sc_spmm_body,
        out_type=jax.ShapeDtypeStruct((2, _N, _H), _f32),
        mesh=mesh,
        scratch_types=[
            pltpu.VMEM_SHARED((_N, _H), _f32),
            pltpu.VMEM((_CHUNK, _H), _f32),
            pltpu.VMEM((_CHUNK, _H), _f32),
            pltpu.VMEM((_SCE,), jnp.int32),
            pltpu.VMEM((_SCE,), jnp.int32),
            pltpu.VMEM((_SCCH, _CHUNK), jnp.int32),
            pltpu.VMEM((_SCCH, _CHUNK), jnp.int32),
            pltpu.VMEM((_SCE,), _f32),
            pltpu.VMEM((_SCE,), _f32),
            pltpu.SemaphoreType.DMA,
            pltpu.SemaphoreType.DMA,
            pltpu.SemaphoreType.DMA,
            pltpu.SemaphoreType.DMA,
        ],
    )
    return fn(y, srcp, dstp, valp)


def _sc_rho_body(src_hbm, dst_hbm, val_hbm, num_out, den_out,
                 shw, shv, shs, vloc, wloc, rbuf, tbuf, sbuf, pbuf,
                 se, de, ve, sem):
    s = lax.axis_index("s")
    zv = jnp.zeros((16,), _f32)
    col = s * _CPT

    pltpu.sync_copy(src_hbm.at[pl.ds(s * _REPT, _REPT)], se)
    pltpu.sync_copy(dst_hbm.at[pl.ds(s * _REPT, _REPT)], de)
    pltpu.sync_copy(val_hbm.at[pl.ds(s * _REPT, _REPT)], ve)

    c0 = jnp.float32(float(_N) ** -0.5)
    lane = lax.iota(jnp.int32, 16)
    def vinit(i, _):
        base = col + 16 * i
        m = (base + lane) < _N
        rbuf[pl.ds(16 * i, 16)] = jnp.where(m, c0, 0.0)
        return 0
    lax.fori_loop(0, _CPT // 16, vinit, 0)
    pltpu.sync_copy(rbuf, shv.at[pl.ds(col, _CPT)])
    plsc.subcore_barrier()
    pltpu.sync_copy(shv, vloc)

    def matvec():
        def z16(i, _):
            wloc[pl.ds(16 * i, 16)] = zv
            return 0
        lax.fori_loop(0, _NP // 16, z16, 0)

        def eg(g, _):
            idx = se[pl.ds(16 * g, 16)]
            vv = plsc.load_gather(vloc, [idx])
            prod = vv * ve[pl.ds(16 * g, 16)]
            plsc.addupdate_scatter(wloc, [de[pl.ds(16 * g, 16)]], prod)
            return 0
        lax.fori_loop(0, _REPT // 16, eg, 0)
        pltpu.sync_copy(wloc, shw.at[s])
        plsc.subcore_barrier()
        pltpu.sync_copy(shw.at[0, pl.ds(col, _CPT)], rbuf)
        for t in range(1, 16):
            pltpu.sync_copy(shw.at[t, pl.ds(col, _CPT)], tbuf)
            def addv(i, _):
                rbuf[pl.ds(16 * i, 16)] = (rbuf[pl.ds(16 * i, 16)]
                                           + tbuf[pl.ds(16 * i, 16)])
                return 0
            lax.fori_loop(0, _CPT // 16, addv, 0)

    def global_sumsq_of_rbuf():
        def sq(i, accv):
            x = rbuf[pl.ds(16 * i, 16)]
            return accv + x * x
        part = lax.fori_loop(0, _CPT // 16, sq, zv)
        pbuf[pl.ds(0, 16)] = part
        pltpu.sync_copy(pbuf, shs.at[s])
        plsc.subcore_barrier()
        pltpu.sync_copy(shs, sbuf)
        def st(t, accv):
            return accv + sbuf[t]
        tot = lax.fori_loop(0, 16, st, zv)
        ss = lax.reduce_sum_p.bind(tot, axes=(0,))
        plsc.subcore_barrier()
        return zv + ss

    def power_iter(it, _):
        matvec()
        ssv = global_sumsq_of_rbuf()
        inv = (zv + jnp.float32(1.0)) / (ssv + jnp.float32(1e-30))
        def nm(i, _):
            rbuf[pl.ds(16 * i, 16)] = rbuf[pl.ds(16 * i, 16)] * inv
            return 0
        lax.fori_loop(0, _CPT // 16, nm, 0)
        pltpu.sync_copy(rbuf, shv.at[pl.ds(col, _CPT)])
        plsc.subcore_barrier()
        pltpu.sync_copy(shv, vloc)
        return 0
    lax.fori_loop(0, _ITERS, power_iter, 0)

    matvec()
    num = global_sumsq_of_rbuf()
    def cpv(i, _):
        rbuf[pl.ds(16 * i, 16)] = vloc[pl.ds(col + 16 * i, 16)]
        return 0
    lax.fori_loop(0, _CPT // 16, cpv, 0)
    den = global_sumsq_of_rbuf()

    @pl.when(s == 0)
    def _():
        pbuf[pl.ds(0, 16)] = num
        pltpu.sync_copy(pbuf, num_out)
        pbuf[pl.ds(0, 16)] = den
        pltpu.sync_copy(pbuf, den_out)


def _sc_rho(src, dst, val):
    mesh = plsc.VectorSubcoreMesh(core_axis_name="c", subcore_axis_name="s",
                                  num_cores=1)
    fn = pl.kernel(
        _sc_rho_body,
        out_type=[jax.ShapeDtypeStruct((16,), _f32),
                  jax.ShapeDtypeStruct((16,), _f32)],
        mesh=mesh,
        scratch_types=[
            pltpu.VMEM_SHARED((16, _NP), _f32),
            pltpu.VMEM_SHARED((_NP,), _f32),
            pltpu.VMEM_SHARED((16, 16), _f32),
            pltpu.VMEM((_NP,), _f32),
            pltpu.VMEM((_NP,), _f32),
            pltpu.VMEM((_CPT,), _f32),
            pltpu.VMEM((_CPT,), _f32),
            pltpu.VMEM((16, 16), _f32),
            pltpu.VMEM((16,), _f32),
            pltpu.VMEM((_REPT,), jnp.int32),
            pltpu.VMEM((_REPT,), jnp.int32),
            pltpu.VMEM((_REPT,), _f32),
            pltpu.SemaphoreType.DMA,
        ],
        compiler_params=pltpu.CompilerParams(needs_layout_passes=False),
    )
    return fn(src, dst, val)


_BLK = 2000
_GRID = _N // _BLK

def _row_spec(w):
    return pl.BlockSpec((_BLK, w), lambda i: (i, 0))

def _full_spec(h, w):
    return pl.BlockSpec((h, w), lambda i: (0, 0))


def _tc_prep_body(u_ref, s0_ref, s1_ref, wi_ref, b_ref, z0_ref, aub_ref):
    z0_ref[...] = jnp.dot(u_ref[...], wi_ref[...],
                          preferred_element_type=_f32)
    aub_ref[...] = jnp.dot(s0_ref[...] + s1_ref[...], b_ref[...],
                           preferred_element_type=_f32)


def _tc_prep(u, s0, s1, w_init, b):
    return pl.pallas_call(
        _tc_prep_body,
        grid=(_GRID,),
        in_specs=[_row_spec(_H), _row_spec(_H), _row_spec(_H),
                  _full_spec(_H, _H), _full_spec(_H, _H)],
        out_specs=[_row_spec(_H), _row_spec(_H)],
        out_shape=[jax.ShapeDtypeStruct((_N, _H), _f32),
                   jax.ShapeDtypeStruct((_N, _H), _f32)],
    )(u, s0, s1, w_init, b)


def _tc_proj_body(w_ref, num_ref, den_ref, wp_ref):
    rho = jnp.sqrt(num_ref[...] / den_ref[...])
    rho = jnp.maximum(rho, jnp.float32(1e-6))
    v = (jnp.float32(_KAPPA) / rho)[0:1, 0:1]
    w = w_ref[...]
    absw = jnp.abs(w)
    rowsum = jnp.sum(absw, axis=1, keepdims=True)
    hi0 = jnp.max(absw, axis=1, keepdims=True)
    lo0 = jnp.zeros_like(hi0)

    def bis(i, carry):
        lo, hi = carry
        mid = 0.5 * (lo + hi)
        g = jnp.sum(jnp.maximum(absw - mid, 0.0), axis=1, keepdims=True)
        pred = g > v
        return jnp.where(pred, mid, lo), jnp.where(pred, hi, mid)
    lo, hi = lax.fori_loop(0, 60, bis, (lo0, hi0))
    theta = 0.5 * (lo + hi)
    wproj = jnp.sign(w) * jnp.maximum(absw - theta, 0.0)
    wp_ref[...] = jnp.where(rowsum > v, wproj, w)


def _tc_proj(w, num, den):
    return pl.pallas_call(
        _tc_proj_body,
        in_specs=[pl.BlockSpec((_H, _H), lambda: (0, 0)),
                  pl.BlockSpec((1, 16), lambda: (0, 0)),
                  pl.BlockSpec((1, 16), lambda: (0, 0))],
        out_specs=pl.BlockSpec((_H, _H), lambda: (0, 0)),
        out_shape=jax.ShapeDtypeStruct((_H, _H), _f32),
    )(w, num, den)


def _tc_mm_body(x_ref, w_ref, y_ref):
    y_ref[...] = jnp.dot(x_ref[...], w_ref[...], preferred_element_type=_f32)


def _tc_mm(x, w):
    return pl.pallas_call(
        _tc_mm_body,
        grid=(_GRID,),
        in_specs=[_row_spec(_H), _full_spec(_H, _H)],
        out_specs=_row_spec(_H),
        out_shape=jax.ShapeDtypeStruct((_N, _H), _f32),
    )(x, w)


def _tc_iter_body(s0_ref, s1_ref, aub_ref, wp_ref, y_ref):
    z = jnp.maximum(s0_ref[...] + s1_ref[...] + aub_ref[...], 0.0)
    y_ref[...] = jnp.dot(z, wp_ref[...], preferred_element_type=_f32)


def _tc_iter(s0, s1, aub, wp):
    return pl.pallas_call(
        _tc_iter_body,
        grid=(_GRID,),
        in_specs=[_row_spec(_H), _row_spec(_H), _row_spec(_H),
                  _full_spec(_H, _H)],
        out_specs=_row_spec(_H),
        out_shape=jax.ShapeDtypeStruct((_N, _H), _f32),
    )(s0, s1, aub, wp)


def _tc_final_body(s0_ref, s1_ref, aub_ref, vt_ref, out_ref):
    z = jnp.maximum(s0_ref[...] + s1_ref[...] + aub_ref[...], 0.0)
    out_ref[...] = jnp.dot(z, vt_ref[...], preferred_element_type=_f32)


def _tc_final(s0, s1, aub, vt):
    nclass = vt.shape[1]
    return pl.pallas_call(
        _tc_final_body,
        grid=(_GRID,),
        in_specs=[_row_spec(_H), _row_spec(_H), _row_spec(_H),
                  _full_spec(_H, nclass)],
        out_specs=_row_spec(nclass),
        out_shape=jax.ShapeDtypeStruct((_N, nclass), _f32),
    )(s0, s1, aub, vt)


def kernel(U, edge_index, edge_values, W, B, W_init, V_w):
    dst = edge_index[0]
    src = edge_index[1]
    npad = _EPAD - _E
    srcp = jnp.concatenate([src, jnp.zeros((npad,), jnp.int32)])
    srcp = srcp.reshape(32, _NSC, _SCE)
    dstp = jnp.concatenate([dst, jnp.zeros((npad,), jnp.int32)])
    dstp = dstp.reshape(32, _NSC, _SCCH, _CHUNK)
    valp = jnp.concatenate([edge_values, jnp.zeros((npad,), _f32)])
    valp = valp.reshape(32, _NSC, _SCE)

    num, den = _sc_rho(src, dst, edge_values)
    su = _sc_spmm(U, srcp, dstp, valp)
    z0, aub = _tc_prep(U, su[0], su[1], W_init, B)
    wp = _tc_proj(W, num.reshape(1, 16), den.reshape(1, 16))

    y = _tc_mm(z0, wp)

    def body(i, y):
        sp = _sc_spmm(y, srcp, dstp, valp)
        return _tc_iter(sp[0], sp[1], aub, wp)
    y = lax.fori_loop(0, _ITERS - 1, body, y)

    sp = _sc_spmm(y, srcp, dstp, valp)
    return _tc_final(sp[0], sp[1], aub, jnp.transpose(V_w))


# speedup vs baseline: 1.0144x; 1.0144x over previous
import jax
import jax.numpy as jnp
from jax import lax
from jax.experimental import pallas as pl
from jax.experimental.pallas import tpu as pltpu
from jax.experimental.pallas import tpu_sc as plsc

_N = 10000
_E = 320000
_H = 128
_KAPPA = 0.99
_ITERS = 30

_CHUNK = 128
_NCH = 80
_EPT = _NCH * _CHUNK
_EPAD = _EPT * 32
_RPT = _N // 16

_NP = 10240
_CPT = _NP // 16
_REPT = _E // 16

_f32 = jnp.float32


def _sc_spmm_body(y_hbm, src_hbm, dst_hbm, val_hbm, out_hbm,
                  acc, ga, gb, sa, sb, da, db, va, vb, sema, semb):
    c = lax.axis_index("c")
    s = lax.axis_index("s")
    wid = c * 16 + s
    zv = jnp.zeros((16,), _f32)

    def zrow(r, _):
        for f in range(8):
            ga[r, 16 * f:16 * (f + 1)] = zv
        return 0
    lax.fori_loop(0, _CHUNK, zrow, 0)
    row0 = s * 632
    nch8 = jnp.where(s < 15, 79, 65)

    def zset(j, _):
        pltpu.sync_copy(ga.at[pl.ds(0, 8)], acc.at[pl.ds(row0 + 8 * j, 8)])
        return 0
    lax.fori_loop(0, nch8, zset, 0)
    plsc.subcore_barrier()

    def weight(gbuf, vbuf):
        def wgrp(g, _):
            val16 = vbuf[pl.ds(16 * g, 16)]
            for lane in range(16):
                e = 16 * g + lane
                v = val16[lane]
                for f in range(8):
                    gbuf[e, 16 * f:16 * (f + 1)] = (
                        gbuf[e, 16 * f:16 * (f + 1)] * v)
            return 0
        lax.fori_loop(0, _CHUNK // 16, wgrp, 0)

    ebase = wid * _EPT

    pltpu.sync_copy(src_hbm.at[pl.ds(ebase, _CHUNK)], sa)
    pltpu.async_copy(y_hbm.at[sa], ga, sema)

    def pair(p, _):
        c0 = 2 * p
        c1 = c0 + 1
        pltpu.sync_copy(src_hbm.at[pl.ds(ebase + c1 * _CHUNK, _CHUNK)], sb)
        pltpu.async_copy(y_hbm.at[sb], gb, semb)
        pltpu.sync_copy(dst_hbm.at[pl.ds(ebase + c0 * _CHUNK, _CHUNK)], da)
        pltpu.sync_copy(val_hbm.at[pl.ds(ebase + c0 * _CHUNK, _CHUNK)], va)
        pltpu.make_async_copy(y_hbm.at[sa], ga, sema).wait()
        weight(ga, va)
        pltpu.sync_copy(ga, acc.at[da], add=True)

        @pl.when(p < _NCH // 2 - 1)
        def _():
            pltpu.sync_copy(src_hbm.at[pl.ds(ebase + (c0 + 2) * _CHUNK,
                                             _CHUNK)], sa)
            pltpu.async_copy(y_hbm.at[sa], ga, sema)
        pltpu.sync_copy(dst_hbm.at[pl.ds(ebase + c1 * _CHUNK, _CHUNK)], db)
        pltpu.sync_copy(val_hbm.at[pl.ds(ebase + c1 * _CHUNK, _CHUNK)], vb)
        pltpu.make_async_copy(y_hbm.at[sb], gb, semb).wait()
        weight(gb, vb)
        pltpu.sync_copy(gb, acc.at[db], add=True)
        return 0
    lax.fori_loop(0, _NCH // 2, pair, 0)
    plsc.subcore_barrier()

    def wout(j, _):
        o = row0 + 8 * j
        pltpu.sync_copy(acc.at[pl.ds(o, 8)], out_hbm.at[c, pl.ds(o, 8)])
        return 0
    lax.fori_loop(0, nch8, wout, 0)


def _sc_spmm(y, srcp, dstp, valp):
    mesh = plsc.VectorSubcoreMesh(core_axis_name="c", subcore_axis_name="s")
    fn = pl.kernel(
        _sc_spmm_body,
        out_type=jax.ShapeDtypeStruct((2, _N, _H), _f32),
        mesh=mesh,
        scratch_types=[
            pltpu.VMEM_SHARED((_N, _H), _f32),
            pltpu.VMEM((_CHUNK, _H), _f32),
            pltpu.VMEM((_CHUNK, _H), _f32),
            pltpu.VMEM((_CHUNK,), jnp.int32),
            pltpu.VMEM((_CHUNK,), jnp.int32),
            pltpu.VMEM((_CHUNK,), jnp.int32),
            pltpu.VMEM((_CHUNK,), jnp.int32),
            pltpu.VMEM((_CHUNK,), _f32),
            pltpu.VMEM((_CHUNK,), _f32),
            pltpu.SemaphoreType.DMA,
            pltpu.SemaphoreType.DMA,
        ],
    )
    return fn(y, srcp, dstp, valp)


def _sc_rho_body(src_hbm, dst_hbm, val_hbm, num_out, den_out,
                 shw, shv, shs, vloc, wloc, rbuf, tbuf, sbuf, pbuf,
                 se, de, ve, sem):
    s = lax.axis_index("s")
    zv = jnp.zeros((16,), _f32)
    col = s * _CPT

    pltpu.sync_copy(src_hbm.at[pl.ds(s * _REPT, _REPT)], se)
    pltpu.sync_copy(dst_hbm.at[pl.ds(s * _REPT, _REPT)], de)
    pltpu.sync_copy(val_hbm.at[pl.ds(s * _REPT, _REPT)], ve)

    c0 = jnp.float32(float(_N) ** -0.5)
    lane = lax.iota(jnp.int32, 16)
    def vinit(i, _):
        base = col + 16 * i
        m = (base + lane) < _N
        rbuf[pl.ds(16 * i, 16)] = jnp.where(m, c0, 0.0)
        return 0
    lax.fori_loop(0, _CPT // 16, vinit, 0)
    pltpu.sync_copy(rbuf, shv.at[pl.ds(col, _CPT)])
    plsc.subcore_barrier()
    pltpu.sync_copy(shv, vloc)

    def matvec():
        def z16(i, _):
            wloc[pl.ds(16 * i, 16)] = zv
            return 0
        lax.fori_loop(0, _NP // 16, z16, 0)

        def eg(g, _):
            idx = se[pl.ds(16 * g, 16)]
            vv = plsc.load_gather(vloc, [idx])
            prod = vv * ve[pl.ds(16 * g, 16)]
            plsc.addupdate_scatter(wloc, [de[pl.ds(16 * g, 16)]], prod)
            return 0
        lax.fori_loop(0, _REPT // 16, eg, 0)
        pltpu.sync_copy(wloc, shw.at[s])
        plsc.subcore_barrier()
        pltpu.sync_copy(shw.at[0, pl.ds(col, _CPT)], rbuf)
        for t in range(1, 16):
            pltpu.sync_copy(shw.at[t, pl.ds(col, _CPT)], tbuf)
            def addv(i, _):
                rbuf[pl.ds(16 * i, 16)] = (rbuf[pl.ds(16 * i, 16)]
                                           + tbuf[pl.ds(16 * i, 16)])
                return 0
            lax.fori_loop(0, _CPT // 16, addv, 0)

    def global_sumsq_of_rbuf():
        def sq(i, accv):
            x = rbuf[pl.ds(16 * i, 16)]
            return accv + x * x
        part = lax.fori_loop(0, _CPT // 16, sq, zv)
        pbuf[pl.ds(0, 16)] = part
        pltpu.sync_copy(pbuf, shs.at[s])
        plsc.subcore_barrier()
        pltpu.sync_copy(shs, sbuf)
        def st(t, accv):
            return accv + sbuf[t]
        tot = lax.fori_loop(0, 16, st, zv)
        ss = lax.reduce_sum_p.bind(tot, axes=(0,))
        plsc.subcore_barrier()
        return zv + ss

    def power_iter(it, _):
        matvec()
        ssv = global_sumsq_of_rbuf()
        inv = (zv + jnp.float32(1.0)) / (ssv + jnp.float32(1e-30))
        def nm(i, _):
            rbuf[pl.ds(16 * i, 16)] = rbuf[pl.ds(16 * i, 16)] * inv
            return 0
        lax.fori_loop(0, _CPT // 16, nm, 0)
        pltpu.sync_copy(rbuf, shv.at[pl.ds(col, _CPT)])
        plsc.subcore_barrier()
        pltpu.sync_copy(shv, vloc)
        return 0
    lax.fori_loop(0, _ITERS, power_iter, 0)

    matvec()
    num = global_sumsq_of_rbuf()
    def cpv(i, _):
        rbuf[pl.ds(16 * i, 16)] = vloc[pl.ds(col + 16 * i, 16)]
        return 0
    lax.fori_loop(0, _CPT // 16, cpv, 0)
    den = global_sumsq_of_rbuf()

    @pl.when(s == 0)
    def _():
        pbuf[pl.ds(0, 16)] = num
        pltpu.sync_copy(pbuf, num_out)
        pbuf[pl.ds(0, 16)] = den
        pltpu.sync_copy(pbuf, den_out)


def _sc_rho(src, dst, val):
    mesh = plsc.VectorSubcoreMesh(core_axis_name="c", subcore_axis_name="s",
                                  num_cores=1)
    fn = pl.kernel(
        _sc_rho_body,
        out_type=[jax.ShapeDtypeStruct((16,), _f32),
                  jax.ShapeDtypeStruct((16,), _f32)],
        mesh=mesh,
        scratch_types=[
            pltpu.VMEM_SHARED((16, _NP), _f32),
            pltpu.VMEM_SHARED((_NP,), _f32),
            pltpu.VMEM_SHARED((16, 16), _f32),
            pltpu.VMEM((_NP,), _f32),
            pltpu.VMEM((_NP,), _f32),
            pltpu.VMEM((_CPT,), _f32),
            pltpu.VMEM((_CPT,), _f32),
            pltpu.VMEM((16, 16), _f32),
            pltpu.VMEM((16,), _f32),
            pltpu.VMEM((_REPT,), jnp.int32),
            pltpu.VMEM((_REPT,), jnp.int32),
            pltpu.VMEM((_REPT,), _f32),
            pltpu.SemaphoreType.DMA,
        ],
        compiler_params=pltpu.CompilerParams(needs_layout_passes=False),
    )
    return fn(src, dst, val)


_BLK = 2000
_GRID = _N // _BLK

def _row_spec(w):
    return pl.BlockSpec((_BLK, w), lambda i: (i, 0))

def _full_spec(h, w):
    return pl.BlockSpec((h, w), lambda i: (0, 0))


def _tc_prep_body(u_ref, s0_ref, s1_ref, wi_ref, b_ref, z0_ref, aub_ref):
    z0_ref[...] = jnp.dot(u_ref[...], wi_ref[...],
                          preferred_element_type=_f32)
    aub_ref[...] = jnp.dot(s0_ref[...] + s1_ref[...], b_ref[...],
                           preferred_element_type=_f32)


def _tc_prep(u, s0, s1, w_init, b):
    return pl.pallas_call(
        _tc_prep_body,
        grid=(_GRID,),
        in_specs=[_row_spec(_H), _row_spec(_H), _row_spec(_H),
                  _full_spec(_H, _H), _full_spec(_H, _H)],
        out_specs=[_row_spec(_H), _row_spec(_H)],
        out_shape=[jax.ShapeDtypeStruct((_N, _H), _f32),
                   jax.ShapeDtypeStruct((_N, _H), _f32)],
    )(u, s0, s1, w_init, b)


def _tc_proj_body(w_ref, num_ref, den_ref, wp_ref):
    rho = jnp.sqrt(num_ref[...] / den_ref[...])
    rho = jnp.maximum(rho, jnp.float32(1e-6))
    v = (jnp.float32(_KAPPA) / rho)[0:1, 0:1]
    w = w_ref[...]
    absw = jnp.abs(w)
    rowsum = jnp.sum(absw, axis=1, keepdims=True)
    hi0 = jnp.max(absw, axis=1, keepdims=True)
    lo0 = jnp.zeros_like(hi0)

    def bis(i, carry):
        lo, hi = carry
        mid = 0.5 * (lo + hi)
        g = jnp.sum(jnp.maximum(absw - mid, 0.0), axis=1, keepdims=True)
        pred = g > v
        return jnp.where(pred, mid, lo), jnp.where(pred, hi, mid)
    lo, hi = lax.fori_loop(0, 60, bis, (lo0, hi0))
    theta = 0.5 * (lo + hi)
    wproj = jnp.sign(w) * jnp.maximum(absw - theta, 0.0)
    wp_ref[...] = jnp.where(rowsum > v, wproj, w)


def _tc_proj(w, num, den):
    return pl.pallas_call(
        _tc_proj_body,
        in_specs=[pl.BlockSpec((_H, _H), lambda: (0, 0)),
                  pl.BlockSpec((1, 16), lambda: (0, 0)),
                  pl.BlockSpec((1, 16), lambda: (0, 0))],
        out_specs=pl.BlockSpec((_H, _H), lambda: (0, 0)),
        out_shape=jax.ShapeDtypeStruct((_H, _H), _f32),
    )(w, num, den)


def _tc_mm_body(x_ref, w_ref, y_ref):
    y_ref[...] = jnp.dot(x_ref[...], w_ref[...], preferred_element_type=_f32)


def _tc_mm(x, w):
    return pl.pallas_call(
        _tc_mm_body,
        grid=(_GRID,),
        in_specs=[_row_spec(_H), _full_spec(_H, _H)],
        out_specs=_row_spec(_H),
        out_shape=jax.ShapeDtypeStruct((_N, _H), _f32),
    )(x, w)


def _tc_iter_body(s0_ref, s1_ref, aub_ref, wp_ref, y_ref):
    z = jnp.maximum(s0_ref[...] + s1_ref[...] + aub_ref[...], 0.0)
    y_ref[...] = jnp.dot(z, wp_ref[...], preferred_element_type=_f32)


def _tc_iter(s0, s1, aub, wp):
    return pl.pallas_call(
        _tc_iter_body,
        grid=(_GRID,),
        in_specs=[_row_spec(_H), _row_spec(_H), _row_spec(_H),
                  _full_spec(_H, _H)],
        out_specs=_row_spec(_H),
        out_shape=jax.ShapeDtypeStruct((_N, _H), _f32),
    )(s0, s1, aub, wp)


def _tc_final_body(s0_ref, s1_ref, aub_ref, vt_ref, out_ref):
    z = jnp.maximum(s0_ref[...] + s1_ref[...] + aub_ref[...], 0.0)
    out_ref[...] = jnp.dot(z, vt_ref[...], preferred_element_type=_f32)


def _tc_final(s0, s1, aub, vt):
    nclass = vt.shape[1]
    return pl.pallas_call(
        _tc_final_body,
        grid=(_GRID,),
        in_specs=[_row_spec(_H), _row_spec(_H), _row_spec(_H),
                  _full_spec(_H, nclass)],
        out_specs=_row_spec(nclass),
        out_shape=jax.ShapeDtypeStruct((_N, nclass), _f32),
    )(s0, s1, aub, vt)


def kernel(U, edge_index, edge_values, W, B, W_init, V_w):
    dst = edge_index[0]
    src = edge_index[1]
    npad = _EPAD - _E
    srcp = jnp.concatenate([src, jnp.zeros((npad,), jnp.int32)])
    dstp = jnp.concatenate([dst, jnp.zeros((npad,), jnp.int32)])
    valp = jnp.concatenate([edge_values, jnp.zeros((npad,), _f32)])

    num, den = _sc_rho(src, dst, edge_values)
    su = _sc_spmm(U, srcp, dstp, valp)
    z0, aub = _tc_prep(U, su[0], su[1], W_init, B)
    wp = _tc_proj(W, num.reshape(1, 16), den.reshape(1, 16))

    y = _tc_mm(z0, wp)

    def body(i, y):
        sp = _sc_spmm(y, srcp, dstp, valp)
        return _tc_iter(sp[0], sp[1], aub, wp)
    y = lax.fori_loop(0, _ITERS - 1, body, y)

    sp = _sc_spmm(y, srcp, dstp, valp)
    return _tc_final(sp[0], sp[1], aub, jnp.transpose(V_w))
